# speedup vs baseline: 11.1267x; 11.1267x over previous
import functools

import jax
import jax.numpy as jnp
from jax import lax
from jax.experimental import pallas as pl
from jax.experimental.pallas import tpu as pltpu
from jax.experimental.pallas import tpu_sc as plsc

VOCAB = 1000000
EMB = 128
WINDOW = 5
HIDDEN = 256
N_TAGS = 50
BATCH = 16384

N_IDX = BATCH * WINDOW
NW = 32
B_PER_W = N_IDX // NW
CHUNK = 512
N_CHUNKS = B_PER_W // CHUNK


def _sc_gather_body(table_hbm, idx_hbm, out_hbm, idx_v, rows_v, sem):
    c = lax.axis_index("c")
    s = lax.axis_index("s")
    wid = s * 2 + c
    base = wid * B_PER_W
    for k in range(N_CHUNKS):
        off = base + k * CHUNK
        pltpu.sync_copy(idx_hbm.at[pl.ds(off, CHUNK)], idx_v)
        pltpu.async_copy(table_hbm.at[idx_v], rows_v, sem).wait()
        pltpu.sync_copy(rows_v, out_hbm.at[pl.ds(off, CHUNK)])


@jax.jit
def _sc_gather(table, idx):
    mesh = plsc.VectorSubcoreMesh(core_axis_name="c", subcore_axis_name="s")
    run = pl.kernel(
        _sc_gather_body,
        mesh=mesh,
        out_type=jax.ShapeDtypeStruct((N_IDX, EMB), jnp.float32),
        scratch_types=[
            pltpu.VMEM((CHUNK,), jnp.int32),
            pltpu.VMEM((CHUNK, EMB), jnp.float32),
            pltpu.SemaphoreType.DMA,
        ],
    )
    return run(table, idx)


def _mlp_body(flat_ref, w1_ref, b1_ref, w2_ref, b2_ref, out_ref):
    h = jnp.tanh(
        jnp.dot(flat_ref[...], w1_ref[...], preferred_element_type=jnp.float32)
        + b1_ref[...]
    )
    out_ref[...] = (
        jnp.dot(h, w2_ref[...], preferred_element_type=jnp.float32) + b2_ref[...]
    )


BM = 1024


@jax.jit
def _mlp(flat, W1, b1, W2p, b2p):
    in_dim = WINDOW * EMB
    return pl.pallas_call(
        _mlp_body,
        grid=(BATCH // BM,),
        in_specs=[
            pl.BlockSpec((BM, in_dim), lambda i: (i, 0)),
            pl.BlockSpec((in_dim, HIDDEN), lambda i: (0, 0)),
            pl.BlockSpec((1, HIDDEN), lambda i: (0, 0)),
            pl.BlockSpec((HIDDEN, 128), lambda i: (0, 0)),
            pl.BlockSpec((1, 128), lambda i: (0, 0)),
        ],
        out_specs=pl.BlockSpec((BM, 128), lambda i: (i, 0)),
        out_shape=jax.ShapeDtypeStruct((BATCH, 128), jnp.float32),
    )(flat, W1, b1, W2p, b2p)


def kernel(x, table, W1, b1, W2, b2):
    idx = x.reshape(-1).astype(jnp.int32)
    rows = _sc_gather(table, idx)
    flat = rows.reshape(BATCH, WINDOW * EMB)
    W2p = jnp.pad(W2, ((0, 0), (0, 128 - N_TAGS)))
    b2p = jnp.pad(b2, (0, 128 - N_TAGS))
    out = _mlp(flat, W1, b1.reshape(1, -1), W2p, b2p.reshape(1, -1))
    return out[:, :N_TAGS]


# speedup vs baseline: 17.8299x; 1.6024x over previous
import functools

import jax
import jax.numpy as jnp
from jax import lax
from jax.experimental import pallas as pl
from jax.experimental.pallas import tpu as pltpu
from jax.experimental.pallas import tpu_sc as plsc

VOCAB = 1000000
EMB = 128
WINDOW = 5
HIDDEN = 256
N_TAGS = 50
BATCH = 16384

N_IDX = BATCH * WINDOW
NW = 32
B_PER_W = N_IDX // NW
CHUNK = 512
N_CHUNKS = B_PER_W // CHUNK


def _sc_gather_body(table_hbm, idx_hbm, out_hbm, idx_v, rows_v, sem):
    c = lax.axis_index("c")
    s = lax.axis_index("s")
    wid = s * 2 + c
    base = wid * B_PER_W
    for k in range(N_CHUNKS):
        off = base + k * CHUNK
        pltpu.sync_copy(idx_hbm.at[pl.ds(off, CHUNK)], idx_v)
        pltpu.async_copy(table_hbm.at[idx_v], rows_v, sem).wait()
        pltpu.sync_copy(rows_v, out_hbm.at[pl.ds(off, CHUNK)])


@jax.jit
def _sc_gather(table, idx):
    mesh = plsc.VectorSubcoreMesh(core_axis_name="c", subcore_axis_name="s")
    run = pl.kernel(
        _sc_gather_body,
        mesh=mesh,
        out_type=jax.ShapeDtypeStruct((N_IDX, EMB), jnp.float32),
        scratch_types=[
            pltpu.VMEM((CHUNK,), jnp.int32),
            pltpu.VMEM((CHUNK, EMB), jnp.float32),
            pltpu.SemaphoreType.DMA,
        ],
    )
    return run(table, idx)


def _mlp_body(rows_ref, w1_ref, b1_ref, w2_ref, b2_ref, out_ref):
    acc = b1_ref[...] + jnp.dot(
        rows_ref[0], w1_ref[0], preferred_element_type=jnp.float32
    )
    for w in range(1, WINDOW):
        acc = acc + jnp.dot(
            rows_ref[w], w1_ref[w], preferred_element_type=jnp.float32
        )
    h = jnp.tanh(acc)
    out_ref[...] = (
        jnp.dot(h, w2_ref[...], preferred_element_type=jnp.float32) + b2_ref[...]
    )


BM = 1024


@jax.jit
def _mlp(rows3, W13, b1, W2p, b2p):
    return pl.pallas_call(
        _mlp_body,
        grid=(BATCH // BM,),
        in_specs=[
            pl.BlockSpec((WINDOW, BM, EMB), lambda i: (0, i, 0)),
            pl.BlockSpec((WINDOW, EMB, HIDDEN), lambda i: (0, 0, 0)),
            pl.BlockSpec((1, HIDDEN), lambda i: (0, 0)),
            pl.BlockSpec((HIDDEN, 128), lambda i: (0, 0)),
            pl.BlockSpec((1, 128), lambda i: (0, 0)),
        ],
        out_specs=pl.BlockSpec((BM, 128), lambda i: (i, 0)),
        out_shape=jax.ShapeDtypeStruct((BATCH, 128), jnp.float32),
    )(rows3, W13, b1, W2p, b2p)


def kernel(x, table, W1, b1, W2, b2):
    idx = x.astype(jnp.int32).T.reshape(-1)
    rows = _sc_gather(table, idx)
    rows3 = rows.reshape(WINDOW, BATCH, EMB)
    W13 = W1.reshape(WINDOW, EMB, HIDDEN)
    W2p = jnp.pad(W2, ((0, 0), (0, 128 - N_TAGS)))
    b2p = jnp.pad(b2, (0, 128 - N_TAGS))
    out = _mlp(rows3, W13, b1.reshape(1, -1), W2p, b2p.reshape(1, -1))
    return out[:, :N_TAGS]


# speedup vs baseline: 18.5710x; 1.0416x over previous
import functools

import jax
import jax.numpy as jnp
from jax import lax
from jax.experimental import pallas as pl
from jax.experimental.pallas import tpu as pltpu
from jax.experimental.pallas import tpu_sc as plsc

VOCAB = 1000000
EMB = 128
WINDOW = 5
HIDDEN = 256
N_TAGS = 50
BATCH = 16384

N_IDX = BATCH * WINDOW
NW = 32
B_PER_W = N_IDX // NW
CHUNK = 320
N_CHUNKS = B_PER_W // CHUNK


def _sc_gather_body(table_hbm, idx_hbm, out_hbm, idx_v, rows0, rows1, sem0, sem1):
    c = lax.axis_index("c")
    s = lax.axis_index("s")
    wid = s * 2 + c
    base = wid * B_PER_W
    pltpu.sync_copy(idx_hbm.at[pl.ds(base, B_PER_W)], idx_v)
    rows = (rows0, rows1)
    sems = (sem0, sem1)
    descs = [None, None]
    descs[0] = pltpu.async_copy(
        table_hbm.at[idx_v.at[pl.ds(0, CHUNK)]], rows[0], sems[0]
    )
    for k in range(N_CHUNKS):
        b = k & 1
        if k + 1 < N_CHUNKS:
            descs[1 - b] = pltpu.async_copy(
                table_hbm.at[idx_v.at[pl.ds((k + 1) * CHUNK, CHUNK)]],
                rows[1 - b],
                sems[1 - b],
            )
        descs[b].wait()
        pltpu.sync_copy(rows[b], out_hbm.at[pl.ds(base + k * CHUNK, CHUNK)])


@jax.jit
def _sc_gather(table, idx):
    mesh = plsc.VectorSubcoreMesh(core_axis_name="c", subcore_axis_name="s")
    run = pl.kernel(
        _sc_gather_body,
        mesh=mesh,
        out_type=jax.ShapeDtypeStruct((N_IDX, EMB), jnp.float32),
        scratch_types=[
            pltpu.VMEM((B_PER_W,), jnp.int32),
            pltpu.VMEM((CHUNK, EMB), jnp.float32),
            pltpu.VMEM((CHUNK, EMB), jnp.float32),
            pltpu.SemaphoreType.DMA,
            pltpu.SemaphoreType.DMA,
        ],
    )
    return run(table, idx)


def _mlp_body(rows_ref, w1_ref, b1_ref, w2_ref, b2_ref, out_ref):
    acc = b1_ref[...] + jnp.dot(
        rows_ref[0], w1_ref[0], preferred_element_type=jnp.float32
    )
    for w in range(1, WINDOW):
        acc = acc + jnp.dot(
            rows_ref[w], w1_ref[w], preferred_element_type=jnp.float32
        )
    h = jnp.tanh(acc)
    out_ref[...] = (
        jnp.dot(h, w2_ref[...], preferred_element_type=jnp.float32) + b2_ref[...]
    )


BM = 1024


@jax.jit
def _mlp(rows3, W13, b1, W2p, b2p):
    return pl.pallas_call(
        _mlp_body,
        grid=(BATCH // BM,),
        in_specs=[
            pl.BlockSpec((WINDOW, BM, EMB), lambda i: (0, i, 0)),
            pl.BlockSpec((WINDOW, EMB, HIDDEN), lambda i: (0, 0, 0)),
            pl.BlockSpec((1, HIDDEN), lambda i: (0, 0)),
            pl.BlockSpec((HIDDEN, 128), lambda i: (0, 0)),
            pl.BlockSpec((1, 128), lambda i: (0, 0)),
        ],
        out_specs=pl.BlockSpec((BM, 128), lambda i: (i, 0)),
        out_shape=jax.ShapeDtypeStruct((BATCH, 128), jnp.float32),
    )(rows3, W13, b1, W2p, b2p)


def kernel(x, table, W1, b1, W2, b2):
    idx = x.astype(jnp.int32).T.reshape(-1)
    rows = _sc_gather(table, idx)
    rows3 = rows.reshape(WINDOW, BATCH, EMB)
    W13 = W1.reshape(WINDOW, EMB, HIDDEN)
    W2p = jnp.pad(W2, ((0, 0), (0, 128 - N_TAGS)))
    b2p = jnp.pad(b2, (0, 128 - N_TAGS))
    out = _mlp(rows3, W13, b1.reshape(1, -1), W2p, b2p.reshape(1, -1))
    return out[:, :N_TAGS]
